# baseline (speedup 1.0000x reference)
import functools
import math

import jax
import jax.numpy as jnp
from jax import lax
from jax.experimental import pallas as pl
from jax.experimental.pallas import tpu as pltpu
from jax.experimental.pallas import tpu_sc as plsc

D_MODEL = 64
SCALE_F = math.sqrt(D_MODEL)
CHUNK = 128
LANES = 16


@functools.partial(jax.jit, static_argnums=(2, 3))
def _embed_lookup(x_flat, embedding, n_rows, n_workers):
    rows_per_w = n_rows // n_workers
    n_chunks = rows_per_w // CHUNK
    mesh = plsc.VectorSubcoreMesh(core_axis_name="c", subcore_axis_name="s")
    num_cores = 2

    @functools.partial(
        pl.kernel,
        mesh=mesh,
        out_type=jax.ShapeDtypeStruct((n_rows, D_MODEL), jnp.float32),
        compiler_params=pltpu.CompilerParams(use_tc_tiling_on_sc=False),
        scratch_types=[
            pltpu.VMEM((CHUNK,), jnp.int32),
            pltpu.VMEM((CHUNK, D_MODEL), jnp.float32),
            pltpu.SemaphoreType.DMA,
        ],
    )
    def k(x_hbm, table_hbm, out_hbm, idx_v, rows_v, sem):
        wid = lax.axis_index("s") * num_cores + lax.axis_index("c")
        base = wid * rows_per_w

        def chunk_body(g, carry):
            off = base + g * CHUNK
            pltpu.sync_copy(x_hbm.at[pl.ds(off, CHUNK)], idx_v)
            pltpu.async_copy(table_hbm.at[idx_v], rows_v, sem).wait()

            def scale_body(r, c2):
                for c in range(D_MODEL // LANES):
                    sl = pl.ds(c * LANES, LANES)
                    rows_v[r, sl] = rows_v[r, sl] * SCALE_F
                return c2

            lax.fori_loop(0, CHUNK, scale_body, 0)
            pltpu.sync_copy(rows_v, out_hbm.at[pl.ds(off, CHUNK)])
            return carry

        lax.fori_loop(0, n_chunks, chunk_body, 0)

    return k(x_flat, embedding)


def kernel(x, embedding):
    b, s = x.shape
    n_rows = b * s
    out = _embed_lookup(x.reshape(n_rows), embedding, n_rows, 32)
    return out.reshape(b, s, D_MODEL)


# speedup vs baseline: 1.2355x; 1.2355x over previous
import functools
import math

import jax
import jax.numpy as jnp
from jax import lax
from jax.experimental import pallas as pl
from jax.experimental.pallas import tpu as pltpu
from jax.experimental.pallas import tpu_sc as plsc

D_MODEL = 64
SCALE_F = math.sqrt(D_MODEL)
CHUNK = 128
K = 4
SUP = CHUNK * K
LANES = 16
NW = 32


@functools.partial(jax.jit, static_argnums=(2,))
def _embed_lookup(x3, embedding, n_rows):
    rows_per_w = n_rows // NW
    n_sup = rows_per_w // SUP
    mesh = plsc.VectorSubcoreMesh(core_axis_name="c", subcore_axis_name="s")
    num_cores = 2

    @functools.partial(
        pl.kernel,
        mesh=mesh,
        out_type=jax.ShapeDtypeStruct((n_rows, D_MODEL), jnp.float32),
        compiler_params=pltpu.CompilerParams(use_tc_tiling_on_sc=False),
        scratch_types=[
            pltpu.VMEM((n_sup, K, CHUNK), jnp.int32),
            pltpu.VMEM((SUP, D_MODEL), jnp.float32),
            pltpu.VMEM((SUP, D_MODEL), jnp.float32),
            pltpu.SemaphoreType.DMA,
            pltpu.SemaphoreType.DMA,
            pltpu.SemaphoreType.DMA,
            pltpu.SemaphoreType.DMA,
        ],
    )
    def k(x_hbm, table_hbm, out_hbm, idx_v, rows0, rows1, g0, g1, o0, o1):
        wid = lax.axis_index("s") * num_cores + lax.axis_index("c")
        base = wid * rows_per_w
        rows = (rows0, rows1)
        gsem = (g0, g1)
        osem = (o0, o1)

        pltpu.sync_copy(x_hbm.at[wid], idx_v)

        def start_gather(j, b):
            for kk in range(K):
                pltpu.async_copy(
                    table_hbm.at[idx_v.at[j, kk]],
                    rows[b].at[pl.ds(kk * CHUNK, CHUNK)],
                    gsem[b],
                )

        def drain_gather(b):
            pltpu.make_async_copy(table_hbm.at[pl.ds(0, SUP)], rows[b], gsem[b]).wait()

        def drain_out(j, b):
            pltpu.make_async_copy(
                rows[b], out_hbm.at[pl.ds(base + j * SUP, SUP)], osem[b]
            ).wait()

        start_gather(0, 0)

        def outer(i, carry):
            j0 = i * 2
            for b in range(2):
                nb = 1 - b
                j = j0 + b

                @pl.when(j >= 1)
                def _():
                    drain_out(j - 1, nb)

                @pl.when(j + 1 < n_sup)
                def _():
                    start_gather(j + 1, nb)

                drain_gather(b)

                def scale_body(r, c2):
                    for c in range(D_MODEL // LANES):
                        sl = pl.ds(c * LANES, LANES)
                        rows[b][r, sl] = rows[b][r, sl] * SCALE_F
                    return c2

                lax.fori_loop(0, SUP, scale_body, 0)

                pltpu.async_copy(
                    rows[b], out_hbm.at[pl.ds(base + j * SUP, SUP)], osem[b]
                )
            return carry

        lax.fori_loop(0, n_sup // 2, outer, 0)
        drain_out(n_sup - 1, (n_sup - 1) % 2)

    return k(x3, embedding)


def kernel(x, embedding):
    b, s = x.shape
    n_rows = b * s
    x3 = x.reshape(NW, n_rows // (NW * K * CHUNK), K, CHUNK)
    out = _embed_lookup(x3, embedding, n_rows)
    return out.reshape(b, s, D_MODEL)


# speedup vs baseline: 1.2555x; 1.0163x over previous
import functools
import math

import jax
import jax.numpy as jnp
from jax import lax
from jax.experimental import pallas as pl
from jax.experimental.pallas import tpu as pltpu
from jax.experimental.pallas import tpu_sc as plsc

D_MODEL = 64
SCALE_F = math.sqrt(D_MODEL)
CHUNK = 128
K = 2
SUP = CHUNK * K
LANES = 16
NW = 32


@functools.partial(jax.jit, static_argnums=(2,))
def _embed_lookup(x3, embedding, n_rows):
    rows_per_w = n_rows // NW
    n_sup = rows_per_w // SUP
    idx_rows = rows_per_w // CHUNK
    out2_per_w = rows_per_w // 2
    mesh = plsc.VectorSubcoreMesh(core_axis_name="c", subcore_axis_name="s")
    num_cores = 2

    @functools.partial(
        pl.kernel,
        mesh=mesh,
        out_type=jax.ShapeDtypeStruct((n_rows // 2, 128), jnp.float32),
        compiler_params=pltpu.CompilerParams(use_tc_tiling_on_sc=False),
        scratch_types=[
            pltpu.VMEM((idx_rows, CHUNK), jnp.int32),
            pltpu.VMEM((SUP, D_MODEL), jnp.float32),
            pltpu.VMEM((SUP, D_MODEL), jnp.float32),
            pltpu.VMEM((SUP // 2, 128), jnp.float32),
            pltpu.VMEM((SUP // 2, 128), jnp.float32),
            pltpu.SemaphoreType.DMA,
            pltpu.SemaphoreType.DMA,
            pltpu.SemaphoreType.DMA,
            pltpu.SemaphoreType.DMA,
        ],
    )
    def k(x_hbm, table_hbm, out_hbm, idx_v, ga, gb, oa, ob, ga_s, gb_s, oa_s, ob_s):
        wid = lax.axis_index("s") * num_cores + lax.axis_index("c")
        base2 = wid * out2_per_w
        gbuf = (ga, gb)
        obuf = (oa, ob)
        gsem = (ga_s, gb_s)
        osem = (oa_s, ob_s)

        pltpu.sync_copy(x_hbm.at[wid], idx_v)

        def start_gather(j, b):
            for kk in range(K):
                pltpu.async_copy(
                    table_hbm.at[idx_v.at[j * K + kk]],
                    gbuf[b].at[pl.ds(kk * CHUNK, CHUNK)],
                    gsem[b],
                )

        def drain_gather(b):
            pltpu.make_async_copy(table_hbm.at[pl.ds(0, SUP)], gbuf[b], gsem[b]).wait()

        def out_copy(j, b):
            return pltpu.make_async_copy(
                obuf[b], out_hbm.at[pl.ds(base2 + j * (SUP // 2), SUP // 2)], osem[b]
            )

        start_gather(0, 0)

        def outer(i, carry):
            j0 = i * 2
            for b in range(2):
                nb = 1 - b
                j = j0 + b

                drain_gather(b)

                @pl.when(j + 1 < n_sup)
                def _():
                    start_gather(j + 1, nb)

                @pl.when(j >= 2)
                def _():
                    out_copy(j - 2, b).wait()

                def scale_body(ro, c2):
                    for co in range(128 // LANES):
                        rg = ro * 2 + co // 4
                        cg = (co % 4) * LANES
                        obuf[b][ro, pl.ds(co * LANES, LANES)] = (
                            gbuf[b][rg, pl.ds(cg, LANES)] * SCALE_F
                        )
                    return c2

                lax.fori_loop(0, SUP // 2, scale_body, 0)

                out_copy(j, b).start()
            return carry

        lax.fori_loop(0, n_sup // 2, outer, 0)
        out_copy(n_sup - 2, 0).wait()
        out_copy(n_sup - 1, 1).wait()

    return k(x3, embedding)


def kernel(x, embedding):
    b, s = x.shape
    n_rows = b * s
    x3 = x.reshape(NW, n_rows // (NW * CHUNK), CHUNK)
    out = _embed_lookup(x3, embedding, n_rows)
    return out.reshape(b, s, D_MODEL)
